# baseline (speedup 1.0000x reference)
import functools

import jax
import jax.numpy as jnp
from jax import lax
from jax.experimental import pallas as pl
from jax.experimental.pallas import tpu as pltpu
from jax.experimental.pallas import tpu_sc as plsc


def _make_gather(S, T, V, D):
    info = plsc.get_sparse_core_info()
    NC, NS = info.num_cores, info.num_subcores
    NW = NC * NS
    L = 16
    SB = S // 128
    n_blocks = T * SB
    bpw = n_blocks // NW
    CB = D // 8
    mesh = plsc.VectorSubcoreMesh(core_axis_name="c", subcore_axis_name="s")

    @functools.partial(
        pl.kernel,
        mesh=mesh,
        out_type=jax.ShapeDtypeStruct((T, CB, SB, 8, 128), jnp.float32),
        scratch_types=[
            [pltpu.VMEM((128,), jnp.int32) for _ in range(2)],
            [pltpu.VMEM((128, D), jnp.float32) for _ in range(2)],
            [pltpu.VMEM((D, 128), jnp.float32) for _ in range(2)],
            [pltpu.SemaphoreType.DMA for _ in range(2)],
            [pltpu.SemaphoreType.DMA for _ in range(2)],
        ],
        compiler_params=pltpu.CompilerParams(
            use_tc_tiling_on_sc=False, needs_layout_passes=False
        ),
    )
    def k(idxT_hbm, table_hbm, out_hbm, idx_b, rows, bufT, sem_g, sem_w):
        wid = lax.axis_index("s") * NC + lax.axis_index("c")
        base = wid * bpw

        def stage_and_gather(blk, b):
            t = blk // SB
            sb = blk % SB
            pltpu.sync_copy(idxT_hbm.at[t, pl.ds(sb * 128, 128)], idx_b[b])
            pltpu.async_copy(table_hbm.at[idx_b[b]], rows[b], sem_g[b])

        def wait_g(b):
            pltpu.make_async_copy(
                table_hbm.at[idx_b[b]], rows[b], sem_g[b]
            ).wait()

        def transpose(b):
            lanes = lax.iota(jnp.int32, L)

            def tr_body(c, carry):
                col = jnp.full((L,), c, jnp.int32)

                def seg_body(s8, carry2):
                    vals = plsc.load_gather(
                        rows[b], [s8 * L + lanes, col]
                    )
                    bufT[b][c, pl.ds(s8 * L, L)] = vals
                    return carry2

                return lax.fori_loop(0, 128 // L, seg_body, carry)

            lax.fori_loop(0, D, tr_body, 0)

        def writeback(blk, b):
            t = blk // SB
            sb = blk % SB
            for cb in range(CB):
                pltpu.async_copy(
                    bufT[b].at[pl.ds(cb * 8, 8)],
                    out_hbm.at[t, cb, sb],
                    sem_w[b],
                )

        def wait_w(blk, b):
            t = blk // SB
            sb = blk % SB
            for cb in range(CB):
                pltpu.make_async_copy(
                    bufT[b].at[pl.ds(cb * 8, 8)],
                    out_hbm.at[t, cb, sb],
                    sem_w[b],
                ).wait()

        stage_and_gather(base, 0)

        def body(j, carry):
            for b in range(2):
                i = 2 * j + b
                blk = base + i

                @pl.when(i + 1 < bpw)
                def _():
                    stage_and_gather(blk + 1, 1 - b)

                wait_g(b)

                @pl.when(i >= 2)
                def _():
                    wait_w(blk - 2, b)

                transpose(b)
                writeback(blk, b)
            return carry

        lax.fori_loop(0, bpw // 2, body, 0)
        wait_w(base + bpw - 2, 0)
        wait_w(base + bpw - 1, 1)

    return k


def kernel(pre_embedding, table):
    S, T = pre_embedding.shape
    V, D = table.shape
    idxT = jnp.transpose(pre_embedding).astype(jnp.int32)
    o5 = _make_gather(S, T, V, D)(idxT, table)
    return jnp.transpose(o5, (2, 4, 0, 1, 3)).reshape(S, T, D)


# speedup vs baseline: 1.0976x; 1.0976x over previous
import functools

import jax
import jax.numpy as jnp
from jax import lax
from jax.experimental import pallas as pl
from jax.experimental.pallas import tpu as pltpu
from jax.experimental.pallas import tpu_sc as plsc

_G = 512
_M = _G // 128


def _make_gather(S, T, V, D):
    info = plsc.get_sparse_core_info()
    NC, NS = info.num_cores, info.num_subcores
    NW = NC * NS
    L = 16
    SB = S // 128
    n_blocks = T * SB
    bpw = n_blocks // NW
    gpw = bpw // _M
    CB = D // 8
    mesh = plsc.VectorSubcoreMesh(core_axis_name="c", subcore_axis_name="s")

    @functools.partial(
        pl.kernel,
        mesh=mesh,
        out_type=jax.ShapeDtypeStruct((T, CB, SB, 8, 128), jnp.float32),
        scratch_types=[
            [pltpu.VMEM((_G,), jnp.int32) for _ in range(2)],
            [pltpu.VMEM((_G, D), jnp.float32) for _ in range(2)],
            [pltpu.VMEM((D, 128), jnp.float32) for _ in range(_M)],
            [pltpu.SemaphoreType.DMA for _ in range(2)],
            [pltpu.SemaphoreType.DMA for _ in range(_M)],
        ],
        compiler_params=pltpu.CompilerParams(
            use_tc_tiling_on_sc=False, needs_layout_passes=False
        ),
    )
    def k(idxF_hbm, table_hbm, out_hbm, idx_b, rows, bufT, sem_g, sem_w):
        wid = lax.axis_index("s") * NC + lax.axis_index("c")
        blk0 = wid * bpw
        lanes = lax.iota(jnp.int32, L)

        def stage_and_gather(g, p):
            pltpu.sync_copy(
                idxF_hbm.at[pl.ds((blk0 + g * _M) * 128, _G)], idx_b[p]
            )
            pltpu.async_copy(table_hbm.at[idx_b[p]], rows[p], sem_g[p])

        def wait_g(p):
            pltpu.make_async_copy(
                table_hbm.at[idx_b[p]], rows[p], sem_g[p]
            ).wait()

        def transpose(p, m):
            svecs = [m * 128 + s8 * L + lanes for s8 in range(128 // L)]

            def tr_body(c, carry):
                cvec = jnp.full((L,), c, jnp.int32)
                for s8 in range(128 // L):
                    vals = plsc.load_gather(rows[p], [svecs[s8], cvec])
                    bufT[m][c, pl.ds(s8 * L, L)] = vals
                return carry

            lax.fori_loop(0, D, tr_body, 0, unroll=2)

        def writeback(blk, m):
            t = blk // SB
            sb = blk % SB
            for cb in range(CB):
                pltpu.async_copy(
                    bufT[m].at[pl.ds(cb * 8, 8)],
                    out_hbm.at[t, cb, sb],
                    sem_w[m],
                )

        def wait_w(m):
            for cb in range(CB):
                pltpu.make_async_copy(
                    bufT[m].at[pl.ds(cb * 8, 8)],
                    out_hbm.at[0, cb, 0],
                    sem_w[m],
                ).wait()

        stage_and_gather(0, 0)

        def body(j, carry):
            for p in range(2):
                g = 2 * j + p

                @pl.when(g + 1 < gpw)
                def _():
                    stage_and_gather(g + 1, 1 - p)

                wait_g(p)
                for m in range(_M):
                    @pl.when(g >= 1)
                    def _():
                        wait_w(m)

                    transpose(p, m)
                    writeback(blk0 + g * _M + m, m)
            return carry

        lax.fori_loop(0, gpw // 2, body, 0)
        for m in range(_M):
            wait_w(m)

    return k


def kernel(pre_embedding, table):
    S, T = pre_embedding.shape
    V, D = table.shape
    idxF = jnp.transpose(pre_embedding).reshape(-1).astype(jnp.int32)
    o5 = _make_gather(S, T, V, D)(idxF, table)
    return jnp.transpose(o5, (2, 4, 0, 1, 3)).reshape(S, T, D)


# speedup vs baseline: 2.7085x; 2.4676x over previous
import functools

import jax
import jax.numpy as jnp
from jax import lax
from jax.experimental import pallas as pl
from jax.experimental.pallas import tpu as pltpu
from jax.experimental.pallas import tpu_sc as plsc

_G = 512
_M = _G // 128


def _make_gather(S, T, V, D):
    info = plsc.get_sparse_core_info()
    NC, NS = info.num_cores, info.num_subcores
    NW = NC * NS
    L = 16
    SB = S // 128
    n_blocks = T * SB
    bpw = n_blocks // NW
    gpw = bpw // _M
    CB = D // 8
    mesh = plsc.VectorSubcoreMesh(core_axis_name="c", subcore_axis_name="s")

    @functools.partial(
        pl.kernel,
        mesh=mesh,
        out_type=jax.ShapeDtypeStruct((T, CB, SB, 8, 128), jnp.float32),
        scratch_types=[
            [pltpu.VMEM((_G,), jnp.int32) for _ in range(2)],
            [pltpu.VMEM((_G, D), jnp.float32) for _ in range(2)],
            [pltpu.VMEM((D, 128), jnp.float32) for _ in range(_M)],
            [pltpu.SemaphoreType.DMA for _ in range(2)],
            [pltpu.SemaphoreType.DMA for _ in range(_M)],
        ],
        compiler_params=pltpu.CompilerParams(
            use_tc_tiling_on_sc=False, needs_layout_passes=False
        ),
    )
    def k(idxF_hbm, table_hbm, out_hbm, idx_b, rows, bufT, sem_g, sem_w):
        wid = lax.axis_index("s") * NC + lax.axis_index("c")
        blk0 = wid * bpw
        lanes = lax.iota(jnp.int32, L)

        def stage_and_gather(g, p):
            pltpu.sync_copy(
                idxF_hbm.at[pl.ds((blk0 + g * _M) * 128, _G)], idx_b[p]
            )
            pltpu.async_copy(table_hbm.at[idx_b[p]], rows[p], sem_g[p])

        def wait_g(p):
            pltpu.make_async_copy(
                table_hbm.at[idx_b[p]], rows[p], sem_g[p]
            ).wait()

        def transpose(p, m):
            svecs = [m * 128 + s8 * L + lanes for s8 in range(128 // L)]

            @functools.partial(plsc.parallel_loop, 0, D, unroll=4)
            def _(c):
                cvec = jnp.full((L,), c, jnp.int32)
                for s8 in range(128 // L):
                    vals = plsc.load_gather(rows[p], [svecs[s8], cvec])
                    bufT[m][c, pl.ds(s8 * L, L)] = vals

        def writeback(blk, m):
            t = blk // SB
            sb = blk % SB
            for cb in range(CB):
                pltpu.async_copy(
                    bufT[m].at[pl.ds(cb * 8, 8)],
                    out_hbm.at[t, cb, sb],
                    sem_w[m],
                )

        def wait_w(m):
            for cb in range(CB):
                pltpu.make_async_copy(
                    bufT[m].at[pl.ds(cb * 8, 8)],
                    out_hbm.at[0, cb, 0],
                    sem_w[m],
                ).wait()

        stage_and_gather(0, 0)

        def body(j, carry):
            for p in range(2):
                g = 2 * j + p

                @pl.when(g + 1 < gpw)
                def _():
                    stage_and_gather(g + 1, 1 - p)

                wait_g(p)
                for m in range(_M):
                    @pl.when(g >= 1)
                    def _():
                        wait_w(m)

                    transpose(p, m)
                    writeback(blk0 + g * _M + m, m)
            return carry

        lax.fori_loop(0, gpw // 2, body, 0)
        for m in range(_M):
            wait_w(m)

    return k


def kernel(pre_embedding, table):
    S, T = pre_embedding.shape
    V, D = table.shape
    idxF = jnp.transpose(pre_embedding).reshape(-1).astype(jnp.int32)
    o5 = _make_gather(S, T, V, D)(idxF, table)
    return jnp.transpose(o5, (2, 4, 0, 1, 3)).reshape(S, T, D)
